# baseline (speedup 1.0000x reference)
import functools

import jax
import jax.numpy as jnp
from jax import lax
from jax.experimental import pallas as pl
from jax.experimental.pallas import tpu as pltpu
from jax.experimental.pallas import tpu_sc as plsc

N = 10000
DEG = 16
F_IN = 128
HID = 64
N_CLS = 10

NC = 2
NS = 16
NW = NC * NS
NP = 10240
PER_W = NP // NW
LANES = 16
LUT = 320

_mesh = plsc.VectorSubcoreMesh(
    core_axis_name="c", subcore_axis_name="s", num_cores=NC, num_subcores=NS)
_sc_params = pltpu.CompilerParams(
    needs_layout_passes=False, use_tc_tiling_on_sc=False)


def _iota():
    return lax.iota(jnp.int32, LANES)


def _full(v):
    return jnp.full((LANES,), v, jnp.int32)


_DNUMS = lax.GatherDimensionNumbers(
    offset_dims=(), collapsed_slice_dims=(0,), start_index_map=(0,))


def _splat_lane(vec, t):
    return lax.gather(vec, _full(t)[:, None], _DNUMS, (1,),
                      mode=lax.GatherScatterMode.PROMISE_IN_BOUNDS)


_SD = 8
_SCH = 32


@functools.partial(
    pl.kernel,
    out_type=(
        jax.ShapeDtypeStruct((NP, 256), jnp.int32),
        jax.ShapeDtypeStruct((NP, DEG), jnp.float32),
        jax.ShapeDtypeStruct((NP, 256), jnp.float32),
        jax.ShapeDtypeStruct((NP,), jnp.float32),
        jax.ShapeDtypeStruct((NP,), jnp.float32),
        jax.ShapeDtypeStruct((NP, HID // 2), jnp.int32),
    ),
    mesh=_mesh,
    compiler_params=_sc_params,
    scratch_types=(
        pltpu.VMEM((NP,), jnp.int32),
        pltpu.VMEM((NP,), jnp.int32),
        pltpu.VMEM((PER_W, DEG), jnp.int32),
        pltpu.VMEM((_SD * LANES, LANES), jnp.int32),
        pltpu.VMEM((2 * _SCH, 256), jnp.int32),
        pltpu.VMEM((2 * _SCH, 256), jnp.float32),
        pltpu.VMEM((PER_W, DEG), jnp.float32),
        pltpu.VMEM((PER_W,), jnp.float32),
        pltpu.VMEM((PER_W,), jnp.float32),
        pltpu.VMEM((LUT,), jnp.float32),
        pltpu.VMEM((_SD * LANES,), jnp.int32),
        pltpu.VMEM((PER_W, HID), jnp.float32),
        pltpu.VMEM((PER_W, HID // 2), jnp.int32),
        pltpu.SemaphoreType.DMA,
        pltpu.SemaphoreType.DMA,
        pltpu.SemaphoreType.DMA,
    ),
)
def _structure_kernel(nbr2d, lut_hbm, r0f, cand_out, w1_out, w2_out,
                      d1_out, d2_out, r0p_out, cnt, slot, nbrslab, cand2d,
                      cslab, w2slab, w1slab, d1slab, d2slab, lutv, idx16,
                      rslab, pslab, semCand, semOutC, semOutW):
    wid = lax.axis_index("s") * NC + lax.axis_index("c")
    base = wid * PER_W
    pltpu.sync_copy(lut_hbm, lutv)
    pltpu.sync_copy(nbr2d.at[pl.ds(base, PER_W)], nbrslab)
    pltpu.sync_copy(r0f.at[pl.ds(base, PER_W)], rslab)

    iota0 = _iota()

    def _pk(li, _):
        row = _full(li)
        for h in range(HID // 32):
            a = plsc.load_gather(rslab, [row, iota0 + 32 * h])
            b = plsc.load_gather(rslab, [row, iota0 + 32 * h + LANES])
            p = plsc.bitcast(
                plsc.pack(a, b, format=plsc.PackFormat.INTERLEAVED),
                jnp.int32)
            plsc.store_scatter(pslab, [row, iota0 + 16 * h], p)
        return 0
    lax.fori_loop(0, PER_W, _pk, 0)
    pltpu.sync_copy(pslab, r0p_out.at[pl.ds(base, PER_W)])

    iota = _iota()
    lane0 = iota == 0
    zeros_i = jnp.zeros((LANES,), jnp.int32)
    ones_i = jnp.ones((LANES,), jnp.int32)

    def _zb(j, _):
        plsc.store_scatter(cnt, [iota + j * LANES], zeros_i)
        return 0
    lax.fori_loop(0, NP // LANES, _zb, 0)

    def issue_cand(li, s):
        v = plsc.load_gather(nbrslab, [_full(li), iota])
        idx16[pl.ds(s * LANES, LANES)] = v
        pltpu.async_copy(nbr2d.at[idx16.at[pl.ds(s * LANES, LANES)]],
                         cand2d.at[pl.ds(s * LANES, LANES)], semCand)

    def wait_cand(s):
        pltpu.make_async_copy(nbr2d.at[idx16.at[pl.ds(s * LANES, LANES)]],
                              cand2d.at[pl.ds(s * LANES, LANES)],
                              semCand).wait()

    def out_slices(s):
        return (cslab.at[pl.ds(s * _SCH, _SCH)],
                w2slab.at[pl.ds(s * _SCH, _SCH)])

    def _prime(li, _):
        issue_cand(li, li)
        return 0
    lax.fori_loop(0, _SD, _prime, 0)

    def chunk(ci, _):
        cslot = lax.rem(ci, 2)
        cs, ws = out_slices(cslot)

        @pl.when(ci >= 2)
        def _():
            pltpu.make_async_copy(cs, cand_out.at[pl.ds(base, _SCH)],
                                  semOutC).wait()
            pltpu.make_async_copy(ws, w2_out.at[pl.ds(base, _SCH)],
                                  semOutW).wait()

        def body(c, _):
            li = ci * _SCH + c
            i = base + li
            i_spl = _full(i)
            v = plsc.load_gather(nbrslab, [_full(li), iota])
            p = lax.rem(li, _SD) * LANES

            plsc.addupdate_scatter(cnt, [v], ones_i)
            g = plsc.load_gather(cnt, [v])
            plsc.store_scatter(slot, [v], iota)
            back = plsc.load_gather(slot, [v])
            chosen = back == iota
            g_adj = g - jnp.where(v == i_spl, 1, 0)
            valid1 = chosen & (g_adj > 0)
            w1v = jnp.where(valid1, 1.0, 0.0)
            plsc.store_scatter(w1slab, [_full(li), iota], w1v)
            deg1 = plsc.all_reduce_population_count(valid1)
            plsc.store_scatter(cnt, [v], zeros_i)

            wait_cand(lax.rem(li, _SD))

            crow = _full(cslot * _SCH + c)
            for s in range(16):
                cv = plsc.load_gather(cand2d, [_full(p + s), iota])
                plsc.store_scatter(cslab, [crow, iota + 16 * s], cv)
                plsc.addupdate_scatter(cnt, [cv], ones_i)
            plsc.addupdate_scatter(cnt, [v], -ones_i)
            plsc.addupdate_scatter(cnt, [i_spl], -ones_i, mask=lane0)
            for s in range(16):
                cv = plsc.load_gather(cand2d, [_full(p + s), iota])
                plsc.store_scatter(slot, [cv], iota + 16 * s)
            deg2 = jnp.zeros((LANES,), jnp.int32)
            for s in range(16):
                cv = plsc.load_gather(cand2d, [_full(p + s), iota])
                g2 = plsc.load_gather(cnt, [cv])
                b2 = plsc.load_gather(slot, [cv])
                m = (b2 == iota + 16 * s) & (g2 > 0)
                plsc.store_scatter(w2slab, [crow, iota + 16 * s],
                                   jnp.where(m, 1.0, 0.0))
                deg2 = deg2 + plsc.all_reduce_population_count(m)
            for s in range(16):
                cv = plsc.load_gather(cand2d, [_full(p + s), iota])
                plsc.store_scatter(cnt, [cv], zeros_i)
            plsc.store_scatter(cnt, [v], zeros_i)
            plsc.store_scatter(cnt, [i_spl], zeros_i, mask=lane0)

            d1s = plsc.load_gather(lutv, [deg1])
            d2s = plsc.load_gather(lutv, [deg2])
            plsc.store_scatter(d1slab, [_full(li)], d1s, mask=lane0)
            plsc.store_scatter(d2slab, [_full(li)], d2s, mask=lane0)

            issue_cand(jnp.minimum(li + _SD, PER_W - 1), lax.rem(li, _SD))
            return 0

        lax.fori_loop(0, _SCH, body, 0)

        pltpu.async_copy(cs, cand_out.at[pl.ds(base + ci * _SCH, _SCH)],
                         semOutC)
        pltpu.async_copy(ws, w2_out.at[pl.ds(base + ci * _SCH, _SCH)],
                         semOutW)
        return 0

    lax.fori_loop(0, PER_W // _SCH, chunk, 0)

    def _drain(j, _):
        wait_cand(lax.rem(j, _SD))
        return 0
    lax.fori_loop(0, _SD, _drain, 0)
    for s in range(2):
        cs, ws = out_slices(s)
        pltpu.make_async_copy(cs, cand_out.at[pl.ds(base, _SCH)],
                              semOutC).wait()
        pltpu.make_async_copy(ws, w2_out.at[pl.ds(base, _SCH)],
                              semOutW).wait()
    pltpu.sync_copy(w1slab, w1_out.at[pl.ds(base, PER_W)])
    pltpu.sync_copy(d1slab, d1_out.at[pl.ds(base, PER_W)])
    pltpu.sync_copy(d2slab, d2_out.at[pl.ds(base, PER_W)])


def _make_layer_kernel(W, pack_out):
    WL = W // LANES
    WP = W // 32

    out_types = [jax.ShapeDtypeStruct((NP, 2 * W), jnp.float32)]
    if pack_out:
        out_types.append(jax.ShapeDtypeStruct((NP, W), jnp.int32))

    @functools.partial(
        pl.kernel,
        out_type=tuple(out_types) if pack_out else out_types[0],
        mesh=_mesh,
        compiler_params=_sc_params,
        scratch_types=(
            pltpu.VMEM((NP,), jnp.float32),
            pltpu.VMEM((NP,), jnp.float32),
            pltpu.VMEM((PER_W, DEG), jnp.int32),
            pltpu.VMEM((PER_W, DEG), jnp.float32),
            pltpu.VMEM((512,), jnp.int32),
            pltpu.VMEM((512,), jnp.float32),
            pltpu.VMEM((272,), jnp.float32),
            pltpu.VMEM((2 * LANES,), jnp.float32),
            pltpu.VMEM((2 * LANES,), jnp.int32),
            pltpu.VMEM((2 * DEG, W // 2), jnp.int32),
            pltpu.VMEM((512, W // 2), jnp.int32),
            pltpu.VMEM((4 * W,), jnp.float32),
            pltpu.VMEM((2 * W,), jnp.int32),
            pltpu.SemaphoreType.DMA,
            pltpu.SemaphoreType.DMA,
            pltpu.SemaphoreType.DMA,
            pltpu.SemaphoreType.DMA,
            pltpu.SemaphoreType.DMA,
            pltpu.SemaphoreType.DMA,
        ),
    )
    def layer(table, nbr2d, cand, w1_in, w2_in, d1_hbm, d2_hbm,
              *outs_and_scratch):
        if pack_out:
            (out_hbm, pout_hbm, d1v, d2v, nbrslab, w1slab, idx256, w2buf,
             coef2, coef1, idx16, rows16, rows256, outbuf, poutbuf, semA,
             semBC, semCand, semW2, semOut, semOutP) = outs_and_scratch
        else:
            (out_hbm, d1v, d2v, nbrslab, w1slab, idx256, w2buf,
             coef2, coef1, idx16, rows16, rows256, outbuf, poutbuf, semA,
             semBC, semCand, semW2, semOut, semOutP) = outs_and_scratch
        wid = lax.axis_index("s") * NC + lax.axis_index("c")
        base = wid * PER_W
        pltpu.sync_copy(d1_hbm, d1v)
        pltpu.sync_copy(d2_hbm, d2v)
        pltpu.sync_copy(nbr2d.at[pl.ds(base, PER_W)], nbrslab)
        pltpu.sync_copy(w1_in.at[pl.ds(base, PER_W)], w1slab)

        iota = _iota()

        def issue_meta(node, s):
            pltpu.async_copy(cand.at[node], idx256.at[pl.ds(s * 256, 256)],
                             semCand)
            pltpu.async_copy(w2_in.at[node], w2buf.at[pl.ds(s * 256, 256)],
                             semW2)

        def wait_meta(s):
            pltpu.make_async_copy(cand.at[base],
                                  idx256.at[pl.ds(s * 256, 256)],
                                  semCand).wait()
            pltpu.make_async_copy(w2_in.at[base],
                                  w2buf.at[pl.ds(s * 256, 256)],
                                  semW2).wait()

        def issue_rows(node_l, s):
            v = plsc.load_gather(nbrslab, [_full(node_l), iota])
            idx16[pl.ds(s * LANES, LANES)] = v
            pltpu.async_copy(table.at[idx16.at[pl.ds(s * LANES, LANES)]],
                             rows16.at[pl.ds(s * DEG, DEG)], semA)
            pltpu.async_copy(
                table.at[idx256.at[pl.ds(s * 256, 128)]],
                rows256.at[pl.ds(s * 256, 128)], semBC)
            pltpu.async_copy(
                table.at[idx256.at[pl.ds(s * 256 + 128, 128)]],
                rows256.at[pl.ds(s * 256 + 128, 128)], semBC)

        def wait_rows(s):
            pltpu.make_async_copy(table.at[idx16.at[pl.ds(s * LANES, LANES)]],
                                  rows16.at[pl.ds(s * DEG, DEG)], semA).wait()
            for h in range(2):
                pltpu.make_async_copy(
                    table.at[idx256.at[pl.ds(s * 256 + h * 128, 128)]],
                    rows256.at[pl.ds(s * 256 + h * 128, 128)], semBC).wait()

        def out_slice(s):
            return outbuf.at[pl.ds(s * 2 * W, 2 * W)]

        def pout_slice(s):
            return poutbuf.at[pl.ds(s * W, W)]

        issue_meta(base, 0)
        issue_meta(base + 1, 1)
        wait_meta(0)
        issue_rows(0, 0)

        def body(li, _):
            po = lax.rem(li, 2)
            pn = lax.rem(li + 1, 2)
            lip1 = jnp.minimum(li + 1, PER_W - 1)
            lip2 = jnp.minimum(li + 2, PER_W - 1)
            i = base + li
            p256 = po * 256

            wait_meta(pn)
            issue_rows(lip1, pn)

            v = plsc.load_gather(nbrslab, [_full(li), iota])
            w1v = plsc.load_gather(w1slab, [_full(li), iota])
            coef1[pl.ds(0, LANES)] = w1v * plsc.load_gather(d1v, [v])

            def cg(g, _):
                cv = idx256[pl.ds(p256 + g * LANES, LANES)]
                coef2[pl.ds(g * LANES, LANES)] = (
                    w2buf[pl.ds(p256 + g * LANES, LANES)]
                    * plsc.load_gather(d2v, [cv]))
                return 0
            lax.fori_loop(0, 16, cg, 0)

            wait_rows(po)

            cols_p = [iota + h * LANES for h in range(WP)]
            U = 8

            def acc_slot(rows_ref, ridx, sp, accs):
                for h in range(WP):
                    pk = plsc.load_gather(rows_ref, [ridx, cols_p[h]])
                    a, b = plsc.unpack(plsc.bitcast(pk, jnp.bfloat16),
                                       format=plsc.PackFormat.INTERLEAVED)
                    accs[2 * h] = accs[2 * h] + sp * a
                    accs[2 * h + 1] = accs[2 * h + 1] + sp * b

            def a1g(g, accs):
                accs = list(accs)
                cvec = coef1[pl.ds(g * U, LANES)]
                for u in range(U):
                    t = g * U + u
                    sp = _splat_lane(cvec, u)
                    acc_slot(rows16, _full(po * DEG + t), sp, accs)
                return tuple(accs)
            acc1 = lax.fori_loop(
                0, DEG // U, a1g,
                tuple(jnp.zeros((LANES,), jnp.float32) for _ in range(WL)))

            def ag(g, accs):
                accs = list(accs)
                cvec = coef2[pl.ds(g * U, LANES)]
                for u in range(U):
                    slot = g * U + u
                    sp = _splat_lane(cvec, u)
                    acc_slot(rows256, _full(p256 + slot), sp, accs)
                return tuple(accs)
            acc2 = lax.fori_loop(
                0, 256 // U, ag,
                tuple(jnp.zeros((LANES,), jnp.float32) for _ in range(WL)))

            @pl.when(li >= 2)
            def _():
                pltpu.make_async_copy(out_slice(po), out_hbm.at[i],
                                      semOut).wait()
                if pack_out:
                    pltpu.make_async_copy(pout_slice(po), pout_hbm.at[i],
                                          semOutP).wait()

            d1i = plsc.load_gather(d1v, [_full(i)])
            d2i = plsc.load_gather(d2v, [_full(i)])
            outs = ([jnp.maximum(acc1[l] * d1i, 0.0) for l in range(WL)]
                    + [jnp.maximum(acc2[l] * d2i, 0.0) for l in range(WL)])
            for l in range(2 * WL):
                outbuf[pl.ds(po * 2 * W + l * LANES, LANES)] = outs[l]
            pltpu.async_copy(out_slice(po), out_hbm.at[i], semOut)
            if pack_out:
                for h in range(WL):
                    p = plsc.bitcast(
                        plsc.pack(outs[2 * h], outs[2 * h + 1],
                                  format=plsc.PackFormat.INTERLEAVED),
                        jnp.int32)
                    poutbuf[pl.ds(po * W + h * LANES, LANES)] = p
                pltpu.async_copy(pout_slice(po), pout_hbm.at[i], semOutP)

            issue_meta(base + lip2, po)
            return 0

        lax.fori_loop(0, PER_W, body, 0)

        wait_meta(0)
        wait_rows(1)
        for s in range(2):
            pltpu.make_async_copy(out_slice(s), out_hbm.at[base], semOut).wait()
            if pack_out:
                pltpu.make_async_copy(pout_slice(s), pout_hbm.at[base],
                                      semOutP).wait()

    return layer


_layer64 = _make_layer_kernel(HID, pack_out=True)
_layer128 = _make_layer_kernel(2 * HID, pack_out=False)


_BK = 1024


def _embed_body(x_ref, w_ref, o_ref):
    o_ref[...] = jnp.maximum(
        jnp.dot(x_ref[...], w_ref[...], preferred_element_type=jnp.float32),
        0.0)


def _classify_body(r0_ref, r1_ref, r2_ref, w_ref, o_ref):
    w = w_ref[...]
    lg = jnp.dot(r0_ref[...], w[0:HID],
                 preferred_element_type=jnp.float32)
    lg = lg + jnp.dot(r1_ref[...], w[HID:3 * HID],
                      preferred_element_type=jnp.float32)
    lg = lg + jnp.dot(r2_ref[...], w[3 * HID:7 * HID],
                      preferred_element_type=jnp.float32)
    m = jnp.max(lg, axis=1, keepdims=True)
    e = jnp.exp(lg - m)
    o_ref[...] = e / jnp.sum(e, axis=1, keepdims=True)


def kernel(x, edge_index, w_embed, w_classify):
    src = edge_index[1].astype(jnp.int32)
    nbr2d = jnp.zeros((NP, DEG), jnp.int32).at[:N].set(src.reshape(N, DEG))
    lut = jnp.where(jnp.arange(LUT) > 0,
                    jnp.arange(LUT, dtype=jnp.float32) ** -0.5,
                    0.0).astype(jnp.float32)
    x_pad = jnp.zeros((NP, F_IN), x.dtype).at[:N].set(x)

    r0 = pl.pallas_call(
        _embed_body,
        grid=(NP // _BK,),
        in_specs=[
            pl.BlockSpec((_BK, F_IN), lambda g: (g, 0)),
            pl.BlockSpec((F_IN, HID), lambda g: (0, 0)),
        ],
        out_specs=pl.BlockSpec((_BK, HID), lambda g: (g, 0)),
        out_shape=jax.ShapeDtypeStruct((NP, HID), jnp.float32),
    )(x_pad, w_embed)

    cand, w1, w2, d1, d2, r0p = _structure_kernel(nbr2d, lut, r0)

    r1, r1p = _layer64(r0p, nbr2d, cand, w1, w2, d1, d2)
    r2 = _layer128(r1p, nbr2d, cand, w1, w2, d1, d2)

    out = pl.pallas_call(
        _classify_body,
        grid=(NP // _BK,),
        in_specs=[
            pl.BlockSpec((_BK, HID), lambda g: (g, 0)),
            pl.BlockSpec((_BK, 2 * HID), lambda g: (g, 0)),
            pl.BlockSpec((_BK, 4 * HID), lambda g: (g, 0)),
            pl.BlockSpec((7 * HID, N_CLS), lambda g: (0, 0)),
        ],
        out_specs=pl.BlockSpec((_BK, N_CLS), lambda g: (g, 0)),
        out_shape=jax.ShapeDtypeStruct((NP, N_CLS), jnp.float32),
    )(r0, r1, r2, w_classify)

    return out[:N]


# speedup vs baseline: 1.0915x; 1.0915x over previous
import functools

import jax
import jax.numpy as jnp
from jax import lax
from jax.experimental import pallas as pl
from jax.experimental.pallas import tpu as pltpu
from jax.experimental.pallas import tpu_sc as plsc

N = 10000
DEG = 16
F_IN = 128
HID = 64
N_CLS = 10

NC = 2
NS = 16
NW = NC * NS
NP = 10240
PER_W = NP // NW
LANES = 16
LUT = 320

_mesh = plsc.VectorSubcoreMesh(
    core_axis_name="c", subcore_axis_name="s", num_cores=NC, num_subcores=NS)
_sc_params = pltpu.CompilerParams(
    needs_layout_passes=False, use_tc_tiling_on_sc=False)


def _iota():
    return lax.iota(jnp.int32, LANES)


def _full(v):
    return jnp.full((LANES,), v, jnp.int32)


_DNUMS = lax.GatherDimensionNumbers(
    offset_dims=(), collapsed_slice_dims=(0,), start_index_map=(0,))


def _splat_lane(vec, t):
    return lax.gather(vec, _full(t)[:, None], _DNUMS, (1,),
                      mode=lax.GatherScatterMode.PROMISE_IN_BOUNDS)


_SD = 8
_SCH = 32


@functools.partial(
    pl.kernel,
    out_type=(
        jax.ShapeDtypeStruct((NP, 256), jnp.int32),
        jax.ShapeDtypeStruct((NP, DEG), jnp.float32),
        jax.ShapeDtypeStruct((NP, 256), jnp.float32),
        jax.ShapeDtypeStruct((NP,), jnp.float32),
        jax.ShapeDtypeStruct((NP,), jnp.float32),
        jax.ShapeDtypeStruct((NP, HID // 2), jnp.int32),
    ),
    mesh=_mesh,
    compiler_params=_sc_params,
    scratch_types=(
        pltpu.VMEM((NP,), jnp.int32),
        pltpu.VMEM((NP,), jnp.int32),
        pltpu.VMEM((PER_W, DEG), jnp.int32),
        pltpu.VMEM((_SD * LANES, LANES), jnp.int32),
        pltpu.VMEM((2 * _SCH, 256), jnp.int32),
        pltpu.VMEM((2 * _SCH, 256), jnp.float32),
        pltpu.VMEM((PER_W, DEG), jnp.float32),
        pltpu.VMEM((PER_W,), jnp.float32),
        pltpu.VMEM((PER_W,), jnp.float32),
        pltpu.VMEM((LUT,), jnp.float32),
        pltpu.VMEM((_SD * LANES,), jnp.int32),
        pltpu.VMEM((PER_W, HID), jnp.float32),
        pltpu.VMEM((PER_W, HID // 2), jnp.int32),
        pltpu.SemaphoreType.DMA,
        pltpu.SemaphoreType.DMA,
        pltpu.SemaphoreType.DMA,
    ),
)
def _structure_kernel(nbr2d, lut_hbm, r0f, cand_out, w1_out, w2_out,
                      d1_out, d2_out, r0p_out, cnt, slot, nbrslab, cand2d,
                      cslab, w2slab, w1slab, d1slab, d2slab, lutv, idx16,
                      rslab, pslab, semCand, semOutC, semOutW):
    wid = lax.axis_index("s") * NC + lax.axis_index("c")
    base = wid * PER_W
    pltpu.sync_copy(lut_hbm, lutv)
    pltpu.sync_copy(nbr2d.at[pl.ds(base, PER_W)], nbrslab)
    pltpu.sync_copy(r0f.at[pl.ds(base, PER_W)], rslab)

    iota0 = _iota()

    def _pk(li, _):
        row = _full(li)
        for h in range(HID // 32):
            a = plsc.load_gather(rslab, [row, iota0 + 32 * h])
            b = plsc.load_gather(rslab, [row, iota0 + 32 * h + LANES])
            p = plsc.bitcast(
                plsc.pack(a, b, format=plsc.PackFormat.INTERLEAVED),
                jnp.int32)
            plsc.store_scatter(pslab, [row, iota0 + 16 * h], p)
        return 0
    lax.fori_loop(0, PER_W, _pk, 0)
    pltpu.sync_copy(pslab, r0p_out.at[pl.ds(base, PER_W)])

    iota = _iota()
    lane0 = iota == 0
    zeros_i = jnp.zeros((LANES,), jnp.int32)
    ones_i = jnp.ones((LANES,), jnp.int32)

    def _zb(j, _):
        plsc.store_scatter(cnt, [iota + j * LANES], zeros_i)
        return 0
    lax.fori_loop(0, NP // LANES, _zb, 0)

    def issue_cand(li, s):
        v = plsc.load_gather(nbrslab, [_full(li), iota])
        idx16[pl.ds(s * LANES, LANES)] = v
        pltpu.async_copy(nbr2d.at[idx16.at[pl.ds(s * LANES, LANES)]],
                         cand2d.at[pl.ds(s * LANES, LANES)], semCand)

    def wait_cand(s):
        pltpu.make_async_copy(nbr2d.at[idx16.at[pl.ds(s * LANES, LANES)]],
                              cand2d.at[pl.ds(s * LANES, LANES)],
                              semCand).wait()

    def out_slices(s):
        return (cslab.at[pl.ds(s * _SCH, _SCH)],
                w2slab.at[pl.ds(s * _SCH, _SCH)])

    def _prime(li, _):
        issue_cand(li, li)
        return 0
    lax.fori_loop(0, _SD, _prime, 0)

    def chunk(ci, _):
        cslot = lax.rem(ci, 2)
        cs, ws = out_slices(cslot)

        @pl.when(ci >= 2)
        def _():
            pltpu.make_async_copy(cs, cand_out.at[pl.ds(base, _SCH)],
                                  semOutC).wait()
            pltpu.make_async_copy(ws, w2_out.at[pl.ds(base, _SCH)],
                                  semOutW).wait()

        def body(c, _):
            li = ci * _SCH + c
            i = base + li
            i_spl = _full(i)
            v = plsc.load_gather(nbrslab, [_full(li), iota])
            p = lax.rem(li, _SD) * LANES

            plsc.addupdate_scatter(cnt, [v], ones_i)
            g = plsc.load_gather(cnt, [v])
            plsc.store_scatter(slot, [v], iota)
            back = plsc.load_gather(slot, [v])
            chosen = back == iota
            g_adj = g - jnp.where(v == i_spl, 1, 0)
            valid1 = chosen & (g_adj > 0)
            w1v = jnp.where(valid1, 1.0, 0.0)
            plsc.store_scatter(w1slab, [_full(li), iota], w1v)
            deg1 = plsc.all_reduce_population_count(valid1)
            plsc.store_scatter(cnt, [v], zeros_i)

            wait_cand(lax.rem(li, _SD))

            crow = _full(cslot * _SCH + c)
            for s in range(16):
                cv = plsc.load_gather(cand2d, [_full(p + s), iota])
                plsc.store_scatter(cslab, [crow, iota + 16 * s], cv)
                plsc.addupdate_scatter(cnt, [cv], ones_i)
            plsc.addupdate_scatter(cnt, [v], -ones_i)
            plsc.addupdate_scatter(cnt, [i_spl], -ones_i, mask=lane0)
            for s in range(16):
                cv = plsc.load_gather(cand2d, [_full(p + s), iota])
                plsc.store_scatter(slot, [cv], iota + 16 * s)
            deg2 = jnp.zeros((LANES,), jnp.int32)
            for s in range(16):
                cv = plsc.load_gather(cand2d, [_full(p + s), iota])
                g2 = plsc.load_gather(cnt, [cv])
                b2 = plsc.load_gather(slot, [cv])
                m = (b2 == iota + 16 * s) & (g2 > 0)
                plsc.store_scatter(w2slab, [crow, iota + 16 * s],
                                   jnp.where(m, 1.0, 0.0))
                deg2 = deg2 + plsc.all_reduce_population_count(m)
            for s in range(16):
                cv = plsc.load_gather(cand2d, [_full(p + s), iota])
                plsc.store_scatter(cnt, [cv], zeros_i)
            plsc.store_scatter(cnt, [v], zeros_i)
            plsc.store_scatter(cnt, [i_spl], zeros_i, mask=lane0)

            d1s = plsc.load_gather(lutv, [deg1])
            d2s = plsc.load_gather(lutv, [deg2])
            plsc.store_scatter(d1slab, [_full(li)], d1s, mask=lane0)
            plsc.store_scatter(d2slab, [_full(li)], d2s, mask=lane0)

            issue_cand(jnp.minimum(li + _SD, PER_W - 1), lax.rem(li, _SD))
            return 0

        lax.fori_loop(0, _SCH, body, 0)

        pltpu.async_copy(cs, cand_out.at[pl.ds(base + ci * _SCH, _SCH)],
                         semOutC)
        pltpu.async_copy(ws, w2_out.at[pl.ds(base + ci * _SCH, _SCH)],
                         semOutW)
        return 0

    lax.fori_loop(0, PER_W // _SCH, chunk, 0)

    def _drain(j, _):
        wait_cand(lax.rem(j, _SD))
        return 0
    lax.fori_loop(0, _SD, _drain, 0)
    for s in range(2):
        cs, ws = out_slices(s)
        pltpu.make_async_copy(cs, cand_out.at[pl.ds(base, _SCH)],
                              semOutC).wait()
        pltpu.make_async_copy(ws, w2_out.at[pl.ds(base, _SCH)],
                              semOutW).wait()
    pltpu.sync_copy(w1slab, w1_out.at[pl.ds(base, PER_W)])
    pltpu.sync_copy(d1slab, d1_out.at[pl.ds(base, PER_W)])
    pltpu.sync_copy(d2slab, d2_out.at[pl.ds(base, PER_W)])


def _make_layer_kernel(W, pack_out):
    WL = W // LANES
    WP = W // 32

    out_types = [jax.ShapeDtypeStruct((NP, 2 * W), jnp.float32)]
    if pack_out:
        out_types.append(jax.ShapeDtypeStruct((NP, W), jnp.int32))

    @functools.partial(
        pl.kernel,
        out_type=tuple(out_types) if pack_out else out_types[0],
        mesh=_mesh,
        compiler_params=_sc_params,
        scratch_types=(
            pltpu.VMEM((NP,), jnp.float32),
            pltpu.VMEM((NP,), jnp.float32),
            pltpu.VMEM((PER_W, DEG), jnp.int32),
            pltpu.VMEM((PER_W, DEG), jnp.float32),
            pltpu.VMEM((512,), jnp.int32),
            pltpu.VMEM((512,), jnp.float32),
            pltpu.VMEM((272,), jnp.float32),
            pltpu.VMEM((2 * LANES,), jnp.float32),
            pltpu.VMEM((2 * LANES,), jnp.int32),
            pltpu.VMEM((2 * DEG, W // 2), jnp.int32),
            pltpu.VMEM((512, W // 2), jnp.int32),
            pltpu.VMEM((4 * W,), jnp.float32),
            pltpu.VMEM((2 * W,), jnp.int32),
            pltpu.SemaphoreType.DMA,
            pltpu.SemaphoreType.DMA,
            pltpu.SemaphoreType.DMA,
            pltpu.SemaphoreType.DMA,
            pltpu.SemaphoreType.DMA,
            pltpu.SemaphoreType.DMA,
        ),
    )
    def layer(table, nbr2d, cand, w1_in, w2_in, d1_hbm, d2_hbm,
              *outs_and_scratch):
        if pack_out:
            (out_hbm, pout_hbm, d1v, d2v, nbrslab, w1slab, idx256, w2buf,
             coef2, coef1, idx16, rows16, rows256, outbuf, poutbuf, semA,
             semBC, semCand, semW2, semOut, semOutP) = outs_and_scratch
        else:
            (out_hbm, d1v, d2v, nbrslab, w1slab, idx256, w2buf,
             coef2, coef1, idx16, rows16, rows256, outbuf, poutbuf, semA,
             semBC, semCand, semW2, semOut, semOutP) = outs_and_scratch
        wid = lax.axis_index("s") * NC + lax.axis_index("c")
        base = wid * PER_W
        pltpu.sync_copy(d1_hbm, d1v)
        pltpu.sync_copy(d2_hbm, d2v)
        pltpu.sync_copy(nbr2d.at[pl.ds(base, PER_W)], nbrslab)
        pltpu.sync_copy(w1_in.at[pl.ds(base, PER_W)], w1slab)

        iota = _iota()

        def issue_meta(node, s):
            pltpu.async_copy(cand.at[node], idx256.at[pl.ds(s * 256, 256)],
                             semCand)
            pltpu.async_copy(w2_in.at[node], w2buf.at[pl.ds(s * 256, 256)],
                             semW2)

        def wait_meta(s):
            pltpu.make_async_copy(cand.at[base],
                                  idx256.at[pl.ds(s * 256, 256)],
                                  semCand).wait()
            pltpu.make_async_copy(w2_in.at[base],
                                  w2buf.at[pl.ds(s * 256, 256)],
                                  semW2).wait()

        def issue_rows(node_l, s):
            v = plsc.load_gather(nbrslab, [_full(node_l), iota])
            idx16[pl.ds(s * LANES, LANES)] = v
            pltpu.async_copy(table.at[idx16.at[pl.ds(s * LANES, LANES)]],
                             rows16.at[pl.ds(s * DEG, DEG)], semA)
            pltpu.async_copy(
                table.at[idx256.at[pl.ds(s * 256, 128)]],
                rows256.at[pl.ds(s * 256, 128)], semBC)
            pltpu.async_copy(
                table.at[idx256.at[pl.ds(s * 256 + 128, 128)]],
                rows256.at[pl.ds(s * 256 + 128, 128)], semBC)

        def wait_rows(s):
            pltpu.make_async_copy(table.at[idx16.at[pl.ds(s * LANES, LANES)]],
                                  rows16.at[pl.ds(s * DEG, DEG)], semA).wait()
            for h in range(2):
                pltpu.make_async_copy(
                    table.at[idx256.at[pl.ds(s * 256 + h * 128, 128)]],
                    rows256.at[pl.ds(s * 256 + h * 128, 128)], semBC).wait()

        def out_slice(s):
            return outbuf.at[pl.ds(s * 2 * W, 2 * W)]

        def pout_slice(s):
            return poutbuf.at[pl.ds(s * W, W)]

        issue_meta(base, 0)
        issue_meta(base + 1, 1)
        wait_meta(0)
        issue_rows(0, 0)

        def body(li, _):
            po = lax.rem(li, 2)
            pn = lax.rem(li + 1, 2)
            lip1 = jnp.minimum(li + 1, PER_W - 1)
            lip2 = jnp.minimum(li + 2, PER_W - 1)
            i = base + li
            p256 = po * 256

            wait_meta(pn)
            issue_rows(lip1, pn)

            v = plsc.load_gather(nbrslab, [_full(li), iota])
            w1v = plsc.load_gather(w1slab, [_full(li), iota])
            coef1[pl.ds(0, LANES)] = w1v * plsc.load_gather(d1v, [v])

            def cg(g, _):
                cv = idx256[pl.ds(p256 + g * LANES, LANES)]
                coef2[pl.ds(g * LANES, LANES)] = (
                    w2buf[pl.ds(p256 + g * LANES, LANES)]
                    * plsc.load_gather(d2v, [cv]))
                return 0
            lax.fori_loop(0, 16, cg, 0)

            wait_rows(po)

            cols_p = [iota + h * LANES for h in range(WP)]
            U = 4

            def acc_slot(rows_ref, ridx, sp, accs):
                for h in range(WP):
                    pk = plsc.load_gather(rows_ref, [ridx, cols_p[h]])
                    a, b = plsc.unpack(plsc.bitcast(pk, jnp.bfloat16),
                                       format=plsc.PackFormat.INTERLEAVED)
                    accs[2 * h] = accs[2 * h] + sp * a
                    accs[2 * h + 1] = accs[2 * h + 1] + sp * b

            def a1g(g, accs):
                accs = list(accs)
                cvec = coef1[pl.ds(g * U, LANES)]
                for u in range(U):
                    t = g * U + u
                    sp = _splat_lane(cvec, u)
                    acc_slot(rows16, _full(po * DEG + t), sp, accs)
                return tuple(accs)
            acc1 = lax.fori_loop(
                0, DEG // U, a1g,
                tuple(jnp.zeros((LANES,), jnp.float32) for _ in range(WL)))

            def ag(g, accs):
                accs = list(accs)
                cvec = coef2[pl.ds(g * U, LANES)]
                for u in range(U):
                    slot = g * U + u
                    sp = _splat_lane(cvec, u)
                    acc_slot(rows256, _full(p256 + slot), sp, accs)
                return tuple(accs)
            acc2 = lax.fori_loop(
                0, 256 // U, ag,
                tuple(jnp.zeros((LANES,), jnp.float32) for _ in range(WL)))

            @pl.when(li >= 2)
            def _():
                pltpu.make_async_copy(out_slice(po), out_hbm.at[i],
                                      semOut).wait()
                if pack_out:
                    pltpu.make_async_copy(pout_slice(po), pout_hbm.at[i],
                                          semOutP).wait()

            d1i = plsc.load_gather(d1v, [_full(i)])
            d2i = plsc.load_gather(d2v, [_full(i)])
            outs = ([jnp.maximum(acc1[l] * d1i, 0.0) for l in range(WL)]
                    + [jnp.maximum(acc2[l] * d2i, 0.0) for l in range(WL)])
            for l in range(2 * WL):
                outbuf[pl.ds(po * 2 * W + l * LANES, LANES)] = outs[l]
            pltpu.async_copy(out_slice(po), out_hbm.at[i], semOut)
            if pack_out:
                for h in range(WL):
                    p = plsc.bitcast(
                        plsc.pack(outs[2 * h], outs[2 * h + 1],
                                  format=plsc.PackFormat.INTERLEAVED),
                        jnp.int32)
                    poutbuf[pl.ds(po * W + h * LANES, LANES)] = p
                pltpu.async_copy(pout_slice(po), pout_hbm.at[i], semOutP)

            issue_meta(base + lip2, po)
            return 0

        lax.fori_loop(0, PER_W, body, 0)

        wait_meta(0)
        wait_rows(1)
        for s in range(2):
            pltpu.make_async_copy(out_slice(s), out_hbm.at[base], semOut).wait()
            if pack_out:
                pltpu.make_async_copy(pout_slice(s), pout_hbm.at[base],
                                      semOutP).wait()

    return layer


_layer64 = _make_layer_kernel(HID, pack_out=True)
_layer128 = _make_layer_kernel(2 * HID, pack_out=False)


_BK = 1024


def _embed_body(x_ref, w_ref, o_ref):
    o_ref[...] = jnp.maximum(
        jnp.dot(x_ref[...], w_ref[...], preferred_element_type=jnp.float32),
        0.0)


def _classify_body(r0_ref, r1_ref, r2_ref, w_ref, o_ref):
    w = w_ref[...]
    lg = jnp.dot(r0_ref[...], w[0:HID],
                 preferred_element_type=jnp.float32)
    lg = lg + jnp.dot(r1_ref[...], w[HID:3 * HID],
                      preferred_element_type=jnp.float32)
    lg = lg + jnp.dot(r2_ref[...], w[3 * HID:7 * HID],
                      preferred_element_type=jnp.float32)
    m = jnp.max(lg, axis=1, keepdims=True)
    e = jnp.exp(lg - m)
    o_ref[...] = e / jnp.sum(e, axis=1, keepdims=True)


def kernel(x, edge_index, w_embed, w_classify):
    src = edge_index[1].astype(jnp.int32)
    nbr2d = jnp.zeros((NP, DEG), jnp.int32).at[:N].set(src.reshape(N, DEG))
    lut = jnp.where(jnp.arange(LUT) > 0,
                    jnp.arange(LUT, dtype=jnp.float32) ** -0.5,
                    0.0).astype(jnp.float32)
    x_pad = jnp.zeros((NP, F_IN), x.dtype).at[:N].set(x)

    r0 = pl.pallas_call(
        _embed_body,
        grid=(NP // _BK,),
        in_specs=[
            pl.BlockSpec((_BK, F_IN), lambda g: (g, 0)),
            pl.BlockSpec((F_IN, HID), lambda g: (0, 0)),
        ],
        out_specs=pl.BlockSpec((_BK, HID), lambda g: (g, 0)),
        out_shape=jax.ShapeDtypeStruct((NP, HID), jnp.float32),
    )(x_pad, w_embed)

    cand, w1, w2, d1, d2, r0p = _structure_kernel(nbr2d, lut, r0)

    r1, r1p = _layer64(r0p, nbr2d, cand, w1, w2, d1, d2)
    r2 = _layer128(r1p, nbr2d, cand, w1, w2, d1, d2)

    out = pl.pallas_call(
        _classify_body,
        grid=(NP // _BK,),
        in_specs=[
            pl.BlockSpec((_BK, HID), lambda g: (g, 0)),
            pl.BlockSpec((_BK, 2 * HID), lambda g: (g, 0)),
            pl.BlockSpec((_BK, 4 * HID), lambda g: (g, 0)),
            pl.BlockSpec((7 * HID, N_CLS), lambda g: (0, 0)),
        ],
        out_specs=pl.BlockSpec((_BK, N_CLS), lambda g: (g, 0)),
        out_shape=jax.ShapeDtypeStruct((NP, N_CLS), jnp.float32),
    )(r0, r1, r2, w_classify)

    return out[:N]
